# Initial kernel scaffold; baseline (speedup 1.0000x reference)
import functools

import jax
import jax.numpy as jnp
from jax import lax
from jax.experimental import pallas as pl
from jax.experimental.pallas import tpu as pltpu
from jax.experimental.pallas import tpu_sc as plsc

N = 50000
E = 800000
EMB = 64
HALF = 32
ROWS = E // 128
NT = 16
TILE_N = N // NT
ZCH = 625
DEGC = 2000

_MESH = plsc.VectorSubcoreMesh(core_axis_name="c", subcore_axis_name="s")


def _tile_row_range(sid):
    start = (sid * ROWS) // NT
    end = ((sid + 1) * ROWS) // NT
    return start, end


def _deg_body(dst2, deg_out, didx, ones_v, zbuf, dacc):
    cid = lax.axis_index("c")
    sid = lax.axis_index("s")

    def fill_ones(i, carry):
        ones_v[pl.ds(i * 16, 16)] = jnp.ones((16,), jnp.float32)
        return carry

    lax.fori_loop(0, 128 // 16, fill_ones, 0)

    def fill_zero(i, carry):
        zbuf[pl.ds(i * 16, 16)] = jnp.zeros((16,), jnp.float32)
        return carry

    lax.fori_loop(0, DEGC // 16, fill_zero, 0)

    def zero_chunk(j, carry):
        @pl.when(j % NT == sid)
        def _():
            pltpu.sync_copy(zbuf, dacc.at[pl.ds(j * DEGC, DEGC)])
        return carry

    lax.fori_loop(0, N // DEGC, zero_chunk, 0)
    plsc.subcore_barrier()

    start, end = _tile_row_range(sid)

    def edge_row(j, carry):
        pltpu.sync_copy(dst2.at[j], didx)
        pltpu.sync_copy(ones_v, dacc.at[didx], add=True)
        return carry

    lax.fori_loop(start, end, edge_row, 0)
    plsc.subcore_barrier()

    def copy_chunk(j, carry):
        @pl.when(jnp.logical_and(j % NT == sid, cid == 0))
        def _():
            pltpu.sync_copy(dacc.at[pl.ds(j * DEGC, DEGC)],
                            deg_out.at[pl.ds(j * DEGC, DEGC)])
        return carry

    lax.fori_loop(0, N // DEGC, copy_chunk, 0)


_deg_call = pl.kernel(
    _deg_body,
    out_type=jax.ShapeDtypeStruct((N,), jnp.float32),
    mesh=_MESH,
    scratch_types=[
        pltpu.VMEM((128,), jnp.int32),
        pltpu.VMEM((128,), jnp.float32),
        pltpu.VMEM((DEGC,), jnp.float32),
        pltpu.VMEM_SHARED((N,), jnp.float32),
    ],
)


def _agg_body(g_hbm, src2, dst2, agg_out, sidx, didx, rows, zbuf, acc, sem):
    cid = lax.axis_index("c")
    sid = lax.axis_index("s")

    def fill_zero(i, carry):
        r = i // 2
        zbuf[r, pl.ds((i % 2) * 16, 16)] = jnp.zeros((16,), jnp.float32)
        return carry

    lax.fori_loop(0, ZCH * 2, fill_zero, 0)

    base = sid * TILE_N

    def zero_chunk(k, carry):
        pltpu.sync_copy(zbuf, acc.at[pl.ds(base + k * ZCH, ZCH)])
        return carry

    lax.fori_loop(0, TILE_N // ZCH, zero_chunk, 0)
    plsc.subcore_barrier()

    start, end = _tile_row_range(sid)

    def edge_row(j, carry):
        pltpu.sync_copy(src2.at[j], sidx)
        pltpu.sync_copy(dst2.at[j], didx)
        pltpu.async_copy(g_hbm.at[cid].at[sidx], rows, sem).wait()
        pltpu.sync_copy(rows, acc.at[didx], add=True)
        return carry

    lax.fori_loop(start, end, edge_row, 0)
    plsc.subcore_barrier()

    def copy_chunk(k, carry):
        sl = pl.ds(base + k * ZCH, ZCH)
        pltpu.sync_copy(acc.at[sl], agg_out.at[cid].at[sl])
        return carry

    lax.fori_loop(0, TILE_N // ZCH, copy_chunk, 0)


_agg_call = pl.kernel(
    _agg_body,
    out_type=jax.ShapeDtypeStruct((2, N, HALF), jnp.float32),
    mesh=_MESH,
    scratch_types=[
        pltpu.VMEM((128,), jnp.int32),
        pltpu.VMEM((128,), jnp.int32),
        pltpu.VMEM((128, HALF), jnp.float32),
        pltpu.VMEM((ZCH, HALF), jnp.float32),
        pltpu.VMEM_SHARED((N, HALF), jnp.float32),
        pltpu.SemaphoreType.DMA,
    ],
)


BLK = 2000
GRID = N // BLK


def _dis_kernel(deg_ref, dis_ref):
    dis_ref[...] = lax.rsqrt(deg_ref[...] + 1.0)


def _mm1_kernel(x_ref, w_ref, dis_ref, g_ref):
    y = jnp.dot(x_ref[...], w_ref[...], preferred_element_type=jnp.float32)
    y = y * dis_ref[...]
    g_ref[...] = jnp.stack([y[:, :HALF], y[:, HALF:]], axis=0)


def _mid_kernel(use_res, use_relu, use_mm, agg_ref, g_ref, dis_ref, b_ref,
                w_ref, res_ref, h_ref, gn_ref):
    dis = dis_ref[...]
    u = jnp.concatenate([agg_ref[0] + g_ref[0], agg_ref[1] + g_ref[1]],
                        axis=1)
    h = dis * u + b_ref[...]
    if use_res:
        h = h + res_ref[...]
    if use_relu:
        h = jnp.maximum(h, 0.0)
    h_ref[...] = h
    if use_mm:
        y = jnp.dot(h, w_ref[...], preferred_element_type=jnp.float32) * dis
        gn_ref[...] = jnp.stack([y[:, :HALF], y[:, HALF:]], axis=0)
    else:
        gn_ref[...] = jnp.zeros((2, BLK, HALF), jnp.float32)


def _tc_dis(deg):
    return pl.pallas_call(
        _dis_kernel,
        out_shape=jax.ShapeDtypeStruct((GRID, BLK), jnp.float32),
        grid=(1,),
        in_specs=[pl.BlockSpec((GRID, BLK), lambda i: (0, 0))],
        out_specs=pl.BlockSpec((GRID, BLK), lambda i: (0, 0)),
    )(deg.reshape(GRID, BLK))


def _tc_mm1(x, W1, dis):
    return pl.pallas_call(
        _mm1_kernel,
        out_shape=jax.ShapeDtypeStruct((2, N, HALF), jnp.float32),
        grid=(GRID,),
        in_specs=[
            pl.BlockSpec((BLK, 8), lambda i: (i, 0)),
            pl.BlockSpec((8, EMB), lambda i: (0, 0)),
            pl.BlockSpec((BLK, 1), lambda i: (i, 0)),
        ],
        out_specs=pl.BlockSpec((2, BLK, HALF), lambda i: (0, i, 0)),
    )(x, W1, dis)


def _tc_mid(agg, g, dis, b, W, res, use_res, use_relu, use_mm):
    body = functools.partial(_mid_kernel, use_res, use_relu, use_mm)
    out_shape = [
        jax.ShapeDtypeStruct((N, EMB), jnp.float32),
        jax.ShapeDtypeStruct((2, N, HALF), jnp.float32),
    ]
    out_specs = [
        pl.BlockSpec((BLK, EMB), lambda i: (i, 0)),
        pl.BlockSpec((2, BLK, HALF), lambda i: (0, i, 0)),
    ]
    return pl.pallas_call(
        body,
        out_shape=out_shape,
        grid=(GRID,),
        in_specs=[
            pl.BlockSpec((2, BLK, HALF), lambda i: (0, i, 0)),
            pl.BlockSpec((2, BLK, HALF), lambda i: (0, i, 0)),
            pl.BlockSpec((BLK, 1), lambda i: (i, 0)),
            pl.BlockSpec((1, EMB), lambda i: (0, 0)),
            pl.BlockSpec((EMB, EMB), lambda i: (0, 0)),
            pl.BlockSpec((BLK, EMB), lambda i: (i, 0)),
        ],
        out_specs=out_specs,
    )(agg, g, dis, b, W, res)


@jax.jit
def kernel(x, edge_index, W1, b1, W2, b2, W3, b3):
    src2 = edge_index[0].reshape(ROWS, 128)
    dst2 = edge_index[1].reshape(ROWS, 128)

    deg = _deg_call(dst2)
    dis = _tc_dis(deg).reshape(N, 1)

    g1 = _tc_mm1(x, W1, dis)
    agg1 = _agg_call(g1, src2, dst2)
    h1, g2 = _tc_mid(agg1, g1, dis, b1.reshape(1, EMB), W2,
                     jnp.zeros((N, EMB), jnp.float32),
                     use_res=False, use_relu=True, use_mm=True)
    agg2 = _agg_call(g2, src2, dst2)
    h2, g3 = _tc_mid(agg2, g2, dis, b2.reshape(1, EMB), W3, h1,
                     use_res=True, use_relu=True, use_mm=True)
    agg3 = _agg_call(g3, src2, dst2)
    out, _ = _tc_mid(agg3, g3, dis, b3.reshape(1, EMB), W3, h2,
                     use_res=True, use_relu=False, use_mm=False)
    return out


# speedup vs baseline: 5.9681x; 5.9681x over previous
import functools

import jax
import jax.numpy as jnp
from jax import lax
from jax.experimental import pallas as pl
from jax.experimental.pallas import tpu as pltpu
from jax.experimental.pallas import tpu_sc as plsc

N = 50000
E = 800000
EMB = 64
Q = 16
NQ = EMB // Q
ROWS = E // 128
NT = 16
ZCH = 1000
CCH = 125
NCC = N // CCH
DEGC = 2000

_MESH = plsc.VectorSubcoreMesh(core_axis_name="c", subcore_axis_name="s")
_SC_PARAMS = pltpu.CompilerParams(use_tc_tiling_on_sc=False)


def _tile_row_range(sid):
    start = (sid * ROWS) // NT
    end = ((sid + 1) * ROWS) // NT
    return start, end


def _deg_body(dst2, deg_out, didx, ones_v, zbuf, dacc):
    cid = lax.axis_index("c")
    sid = lax.axis_index("s")

    def fill_ones(i, carry):
        ones_v[pl.ds(i * 16, 16)] = jnp.ones((16,), jnp.float32)
        return carry

    lax.fori_loop(0, 128 // 16, fill_ones, 0)

    def fill_zero(i, carry):
        zbuf[pl.ds(i * 16, 16)] = jnp.zeros((16,), jnp.float32)
        return carry

    lax.fori_loop(0, DEGC // 16, fill_zero, 0)

    def zero_chunk(j, carry):
        @pl.when(j % NT == sid)
        def _():
            pltpu.sync_copy(zbuf, dacc.at[pl.ds(j * DEGC, DEGC)])
        return carry

    lax.fori_loop(0, N // DEGC, zero_chunk, 0)
    plsc.subcore_barrier()

    start, end = _tile_row_range(sid)

    def edge_row(j, carry):
        pltpu.sync_copy(dst2.at[j], didx)
        pltpu.sync_copy(ones_v, dacc.at[didx], add=True)
        return carry

    lax.fori_loop(start, end, edge_row, 0)
    plsc.subcore_barrier()

    def copy_chunk(j, carry):
        @pl.when(jnp.logical_and(j % NT == sid, cid == 0))
        def _():
            pltpu.sync_copy(dacc.at[pl.ds(j * DEGC, DEGC)], zbuf)
            pltpu.sync_copy(zbuf, deg_out.at[pl.ds(j * DEGC, DEGC)])
        return carry

    lax.fori_loop(0, N // DEGC, copy_chunk, 0)


_deg_call = pl.kernel(
    _deg_body,
    out_type=jax.ShapeDtypeStruct((N,), jnp.float32),
    mesh=_MESH,
    compiler_params=_SC_PARAMS,
    scratch_types=[
        pltpu.VMEM((128,), jnp.int32),
        pltpu.VMEM((128,), jnp.float32),
        pltpu.VMEM((DEGC,), jnp.float32),
        pltpu.VMEM_SHARED((N,), jnp.float32),
    ],
)


def _agg_body(g4, src4, dst2, oidx4, agg_out,
              sidx, didx, oidx, rows, zbuf, cbuf, acc, sem):
    cid = lax.axis_index("c")
    sid = lax.axis_index("s")

    def fill_zero(i, carry):
        zbuf[i, :] = jnp.zeros((Q,), jnp.float32)
        return carry

    lax.fori_loop(0, ZCH, fill_zero, 0)

    start, end = _tile_row_range(sid)

    for q in range(2):
        tq = cid * 2 + q

        def zero_chunk(j, carry):
            @pl.when(j % NT == sid)
            def _():
                pltpu.sync_copy(zbuf, acc.at[pl.ds(j * ZCH, ZCH)])
            return carry

        lax.fori_loop(0, N // ZCH, zero_chunk, 0)
        plsc.subcore_barrier()

        def edge_row(j, carry):
            pltpu.sync_copy(src4.at[tq].at[j], sidx)
            pltpu.sync_copy(dst2.at[j], didx)
            pltpu.async_copy(g4.at[sidx], rows, sem).wait()
            pltpu.sync_copy(rows, acc.at[didx], add=True)
            return carry

        lax.fori_loop(start, end, edge_row, 0)
        plsc.subcore_barrier()

        def copy_chunk(j, carry):
            @pl.when(j % NT == sid)
            def _():
                pltpu.sync_copy(acc.at[pl.ds(j * CCH, CCH)], cbuf)
                pltpu.sync_copy(oidx4.at[tq].at[j], oidx)
                pltpu.sync_copy(cbuf, agg_out.at[oidx])
            return carry

        lax.fori_loop(0, NCC, copy_chunk, 0)
        plsc.subcore_barrier()


_agg_call = pl.kernel(
    _agg_body,
    out_type=jax.ShapeDtypeStruct((NQ * N, Q), jnp.float32),
    mesh=_MESH,
    compiler_params=_SC_PARAMS,
    scratch_types=[
        pltpu.VMEM((128,), jnp.int32),
        pltpu.VMEM((128,), jnp.int32),
        pltpu.VMEM((CCH,), jnp.int32),
        pltpu.VMEM((128, Q), jnp.float32),
        pltpu.VMEM((ZCH, Q), jnp.float32),
        pltpu.VMEM((CCH, Q), jnp.float32),
        pltpu.VMEM_SHARED((N, Q), jnp.float32),
        pltpu.SemaphoreType.DMA,
    ],
)


BLK = 2000
GRID = N // BLK


def _dis_kernel(deg_ref, dis_ref):
    dis_ref[...] = lax.rsqrt(deg_ref[...] + 1.0)


def _tc_dis(deg):
    return pl.pallas_call(
        _dis_kernel,
        out_shape=jax.ShapeDtypeStruct((GRID, BLK), jnp.float32),
        grid=(1,),
        in_specs=[pl.BlockSpec((GRID, BLK), lambda i: (0, 0))],
        out_specs=pl.BlockSpec((GRID, BLK), lambda i: (0, 0)),
    )(deg.reshape(GRID, BLK))


def _src4_kernel(src_ref, out_ref):
    s = src_ref[...] * 4
    out_ref[...] = jnp.stack([s, s + 1, s + 2, s + 3], axis=0)


def _tc_src4(src2):
    return pl.pallas_call(
        _src4_kernel,
        out_shape=jax.ShapeDtypeStruct((NQ, ROWS, 128), jnp.int32),
        grid=(1,),
        in_specs=[pl.BlockSpec((ROWS, 128), lambda i: (0, 0))],
        out_specs=pl.BlockSpec((NQ, ROWS, 128), lambda i: (0, 0, 0)),
    )(src2)


def _oidx_kernel(out_ref):
    qi = lax.broadcasted_iota(jnp.int32, (NQ, NCC, CCH), 0)
    ri = lax.broadcasted_iota(jnp.int32, (NQ, NCC, CCH), 1)
    ci = lax.broadcasted_iota(jnp.int32, (NQ, NCC, CCH), 2)
    out_ref[...] = 4 * (ri * CCH + ci) + qi


def _tc_oidx():
    return pl.pallas_call(
        _oidx_kernel,
        out_shape=jax.ShapeDtypeStruct((NQ, NCC, CCH), jnp.int32),
        grid=(1,),
        out_specs=pl.BlockSpec((NQ, NCC, CCH), lambda i: (0, 0, 0)),
    )()


def _mm1_kernel(x_ref, w_ref, dis_ref, g_ref):
    y = jnp.dot(x_ref[...], w_ref[...], preferred_element_type=jnp.float32)
    g_ref[...] = y * dis_ref[...]


def _tc_mm1(x, W1, dis):
    return pl.pallas_call(
        _mm1_kernel,
        out_shape=jax.ShapeDtypeStruct((N, EMB), jnp.float32),
        grid=(GRID,),
        in_specs=[
            pl.BlockSpec((BLK, 8), lambda i: (i, 0)),
            pl.BlockSpec((8, EMB), lambda i: (0, 0)),
            pl.BlockSpec((BLK, 1), lambda i: (i, 0)),
        ],
        out_specs=pl.BlockSpec((BLK, EMB), lambda i: (i, 0)),
    )(x, W1, dis)


def _mid_kernel(use_res, use_relu, use_mm, agg_ref, g_ref, dis_ref, b_ref,
                w_ref, res_ref, h_ref, gn_ref):
    dis = dis_ref[...]
    h = dis * (agg_ref[...] + g_ref[...]) + b_ref[...]
    if use_res:
        h = h + res_ref[...]
    if use_relu:
        h = jnp.maximum(h, 0.0)
    h_ref[...] = h
    if use_mm:
        gn_ref[...] = jnp.dot(h, w_ref[...],
                              preferred_element_type=jnp.float32) * dis
    else:
        gn_ref[...] = jnp.zeros((BLK, EMB), jnp.float32)


def _tc_mid(agg, g, dis, b, W, res, use_res, use_relu, use_mm):
    body = functools.partial(_mid_kernel, use_res, use_relu, use_mm)
    return pl.pallas_call(
        body,
        out_shape=[
            jax.ShapeDtypeStruct((N, EMB), jnp.float32),
            jax.ShapeDtypeStruct((N, EMB), jnp.float32),
        ],
        grid=(GRID,),
        in_specs=[
            pl.BlockSpec((BLK, EMB), lambda i: (i, 0)),
            pl.BlockSpec((BLK, EMB), lambda i: (i, 0)),
            pl.BlockSpec((BLK, 1), lambda i: (i, 0)),
            pl.BlockSpec((1, EMB), lambda i: (0, 0)),
            pl.BlockSpec((EMB, EMB), lambda i: (0, 0)),
            pl.BlockSpec((BLK, EMB), lambda i: (i, 0)),
        ],
        out_specs=[
            pl.BlockSpec((BLK, EMB), lambda i: (i, 0)),
            pl.BlockSpec((BLK, EMB), lambda i: (i, 0)),
        ],
    )(agg, g, dis, b, W, res)


@jax.jit
def kernel(x, edge_index, W1, b1, W2, b2, W3, b3):
    src2 = edge_index[0].reshape(ROWS, 128)
    dst2 = edge_index[1].reshape(ROWS, 128)

    deg = _deg_call(dst2)
    dis = _tc_dis(deg).reshape(N, 1)
    src4 = _tc_src4(src2)
    oidx4 = _tc_oidx()

    def agg(g):
        a4 = _agg_call(g.reshape(NQ * N, Q), src4, dst2, oidx4)
        return a4.reshape(N, EMB)

    g1 = _tc_mm1(x, W1, dis)
    agg1 = agg(g1)
    h1, g2 = _tc_mid(agg1, g1, dis, b1.reshape(1, EMB), W2,
                     jnp.zeros((N, EMB), jnp.float32),
                     use_res=False, use_relu=True, use_mm=True)
    agg2 = agg(g2)
    h2, g3 = _tc_mid(agg2, g2, dis, b2.reshape(1, EMB), W3, h1,
                     use_res=True, use_relu=True, use_mm=True)
    agg3 = agg(g3)
    out, _ = _tc_mid(agg3, g3, dis, b3.reshape(1, EMB), W3, h2,
                     use_res=True, use_relu=False, use_mm=False)
    return out


# speedup vs baseline: 17.7642x; 2.9765x over previous
import functools

import jax
import jax.numpy as jnp
from jax import lax
from jax.experimental import pallas as pl
from jax.experimental.pallas import tpu as pltpu
from jax.experimental.pallas import tpu_sc as plsc

N = 50000
E = 800000
EMB = 64
Q = 16
NQ = EMB // Q
ROWS = E // 128
NT = 16
ZCH = 1000
CCH = 125
NCC = N // CCH
DEGC = 2000
EC = 1000

_MESH = plsc.VectorSubcoreMesh(core_axis_name="c", subcore_axis_name="s")
_SC_PARAMS = pltpu.CompilerParams(use_tc_tiling_on_sc=False)


def _tile_row_range(sid):
    start = (sid * ROWS) // NT
    end = ((sid + 1) * ROWS) // NT
    return start, end


def _deg_body(dstf, deg_out, didx, ones_v, zbuf, dacc):
    cid = lax.axis_index("c")
    sid = lax.axis_index("s")

    def fill_ones(i, carry):
        ones_v[pl.ds(i * 16, 16)] = jnp.ones((16,), jnp.float32)
        return carry

    lax.fori_loop(0, DEGC // 16, fill_ones, 0)

    def fill_zero(i, carry):
        zbuf[pl.ds(i * 16, 16)] = jnp.zeros((16,), jnp.float32)
        return carry

    lax.fori_loop(0, DEGC // 16, fill_zero, 0)

    def zero_chunk(j, carry):
        @pl.when(j % NT == sid)
        def _():
            pltpu.sync_copy(zbuf, dacc.at[pl.ds(j * DEGC, DEGC)])
        return carry

    lax.fori_loop(0, N // DEGC, zero_chunk, 0)
    plsc.subcore_barrier()

    ebase = sid * (E // NT)

    def edge_chunk(i, carry):
        pltpu.sync_copy(dstf.at[pl.ds(ebase + i * DEGC, DEGC)], didx)
        pltpu.sync_copy(ones_v, dacc.at[didx], add=True)
        return carry

    lax.fori_loop(0, E // NT // DEGC, edge_chunk, 0)
    plsc.subcore_barrier()

    def copy_chunk(j, carry):
        @pl.when(jnp.logical_and(j % NT == sid, cid == 0))
        def _():
            pltpu.sync_copy(dacc.at[pl.ds(j * DEGC, DEGC)], zbuf)
            pltpu.sync_copy(zbuf, deg_out.at[pl.ds(j * DEGC, DEGC)])
        return carry

    lax.fori_loop(0, N // DEGC, copy_chunk, 0)


_deg_call = pl.kernel(
    _deg_body,
    out_type=jax.ShapeDtypeStruct((N,), jnp.float32),
    mesh=_MESH,
    compiler_params=_SC_PARAMS,
    scratch_types=[
        pltpu.VMEM((DEGC,), jnp.int32),
        pltpu.VMEM((DEGC,), jnp.float32),
        pltpu.VMEM((DEGC,), jnp.float32),
        pltpu.VMEM_SHARED((N,), jnp.float32),
    ],
)


def _agg_body(g4, src4, dstf, oidx4, agg_out,
              sidx, didx, oidx, rows, zbuf, cbuf, acc, sem):
    cid = lax.axis_index("c")
    sid = lax.axis_index("s")

    def fill_zero(i, carry):
        zbuf[i, :] = jnp.zeros((Q,), jnp.float32)
        return carry

    lax.fori_loop(0, ZCH, fill_zero, 0)

    ebase = sid * (E // NT)

    for q in range(2):
        tq = cid * 2 + q

        def zero_chunk(j, carry):
            @pl.when(j % NT == sid)
            def _():
                pltpu.sync_copy(zbuf, acc.at[pl.ds(j * ZCH, ZCH)])
            return carry

        lax.fori_loop(0, N // ZCH, zero_chunk, 0)
        plsc.subcore_barrier()

        def edge_chunk(i, carry):
            sl = pl.ds(ebase + i * EC, EC)
            pltpu.sync_copy(src4.at[tq].at[sl], sidx)
            pltpu.sync_copy(dstf.at[sl], didx)
            pltpu.async_copy(g4.at[sidx], rows, sem).wait()
            pltpu.sync_copy(rows, acc.at[didx], add=True)
            return carry

        lax.fori_loop(0, E // NT // EC, edge_chunk, 0)
        plsc.subcore_barrier()

        def copy_chunk(j, carry):
            @pl.when(j % NT == sid)
            def _():
                pltpu.sync_copy(acc.at[pl.ds(j * CCH, CCH)], cbuf)
                pltpu.sync_copy(oidx4.at[tq].at[j], oidx)
                pltpu.sync_copy(cbuf, agg_out.at[oidx])
            return carry

        lax.fori_loop(0, NCC, copy_chunk, 0)
        plsc.subcore_barrier()


_agg_call = pl.kernel(
    _agg_body,
    out_type=jax.ShapeDtypeStruct((NQ * N, Q), jnp.float32),
    mesh=_MESH,
    compiler_params=_SC_PARAMS,
    scratch_types=[
        pltpu.VMEM((EC,), jnp.int32),
        pltpu.VMEM((EC,), jnp.int32),
        pltpu.VMEM((CCH,), jnp.int32),
        pltpu.VMEM((EC, Q), jnp.float32),
        pltpu.VMEM((ZCH, Q), jnp.float32),
        pltpu.VMEM((CCH, Q), jnp.float32),
        pltpu.VMEM_SHARED((N, Q), jnp.float32),
        pltpu.SemaphoreType.DMA,
    ],
)


BLK = 2000
GRID = N // BLK


def _dis_kernel(deg_ref, dis_ref):
    dis_ref[...] = lax.rsqrt(deg_ref[...] + 1.0)


def _tc_dis(deg):
    return pl.pallas_call(
        _dis_kernel,
        out_shape=jax.ShapeDtypeStruct((GRID, BLK), jnp.float32),
        grid=(1,),
        in_specs=[pl.BlockSpec((GRID, BLK), lambda i: (0, 0))],
        out_specs=pl.BlockSpec((GRID, BLK), lambda i: (0, 0)),
    )(deg.reshape(GRID, BLK))


def _src4_kernel(src_ref, out_ref):
    s = src_ref[...] * 4
    out_ref[...] = jnp.stack([s, s + 1, s + 2, s + 3], axis=0)


def _tc_src4(src2):
    return pl.pallas_call(
        _src4_kernel,
        out_shape=jax.ShapeDtypeStruct((NQ, ROWS, 128), jnp.int32),
        grid=(1,),
        in_specs=[pl.BlockSpec((ROWS, 128), lambda i: (0, 0))],
        out_specs=pl.BlockSpec((NQ, ROWS, 128), lambda i: (0, 0, 0)),
    )(src2)


def _oidx_kernel(out_ref):
    qi = lax.broadcasted_iota(jnp.int32, (NQ, NCC, CCH), 0)
    ri = lax.broadcasted_iota(jnp.int32, (NQ, NCC, CCH), 1)
    ci = lax.broadcasted_iota(jnp.int32, (NQ, NCC, CCH), 2)
    out_ref[...] = 4 * (ri * CCH + ci) + qi


def _tc_oidx():
    return pl.pallas_call(
        _oidx_kernel,
        out_shape=jax.ShapeDtypeStruct((NQ, NCC, CCH), jnp.int32),
        grid=(1,),
        out_specs=pl.BlockSpec((NQ, NCC, CCH), lambda i: (0, 0, 0)),
    )()


def _mm1_kernel(x_ref, w_ref, dis_ref, g_ref):
    y = jnp.dot(x_ref[...], w_ref[...], preferred_element_type=jnp.float32)
    g_ref[...] = y * dis_ref[...]


def _tc_mm1(x, W1, dis):
    return pl.pallas_call(
        _mm1_kernel,
        out_shape=jax.ShapeDtypeStruct((N, EMB), jnp.float32),
        grid=(GRID,),
        in_specs=[
            pl.BlockSpec((BLK, 8), lambda i: (i, 0)),
            pl.BlockSpec((8, EMB), lambda i: (0, 0)),
            pl.BlockSpec((BLK, 1), lambda i: (i, 0)),
        ],
        out_specs=pl.BlockSpec((BLK, EMB), lambda i: (i, 0)),
    )(x, W1, dis)


def _mid_kernel(use_res, use_relu, use_mm, agg_ref, g_ref, dis_ref, b_ref,
                w_ref, res_ref, h_ref, gn_ref):
    dis = dis_ref[...]
    h = dis * (agg_ref[...] + g_ref[...]) + b_ref[...]
    if use_res:
        h = h + res_ref[...]
    if use_relu:
        h = jnp.maximum(h, 0.0)
    h_ref[...] = h
    if use_mm:
        gn_ref[...] = jnp.dot(h, w_ref[...],
                              preferred_element_type=jnp.float32) * dis
    else:
        gn_ref[...] = jnp.zeros((BLK, EMB), jnp.float32)


def _tc_mid(agg, g, dis, b, W, res, use_res, use_relu, use_mm):
    body = functools.partial(_mid_kernel, use_res, use_relu, use_mm)
    return pl.pallas_call(
        body,
        out_shape=[
            jax.ShapeDtypeStruct((N, EMB), jnp.float32),
            jax.ShapeDtypeStruct((N, EMB), jnp.float32),
        ],
        grid=(GRID,),
        in_specs=[
            pl.BlockSpec((BLK, EMB), lambda i: (i, 0)),
            pl.BlockSpec((BLK, EMB), lambda i: (i, 0)),
            pl.BlockSpec((BLK, 1), lambda i: (i, 0)),
            pl.BlockSpec((1, EMB), lambda i: (0, 0)),
            pl.BlockSpec((EMB, EMB), lambda i: (0, 0)),
            pl.BlockSpec((BLK, EMB), lambda i: (i, 0)),
        ],
        out_specs=[
            pl.BlockSpec((BLK, EMB), lambda i: (i, 0)),
            pl.BlockSpec((BLK, EMB), lambda i: (i, 0)),
        ],
    )(agg, g, dis, b, W, res)


@jax.jit
def kernel(x, edge_index, W1, b1, W2, b2, W3, b3):
    src2 = edge_index[0].reshape(ROWS, 128)
    dstf = edge_index[1]

    deg = _deg_call(dstf)
    dis = _tc_dis(deg).reshape(N, 1)
    src4 = _tc_src4(src2).reshape(NQ, E)
    oidx4 = _tc_oidx()

    def agg(g):
        a4 = _agg_call(g.reshape(NQ * N, Q), src4, dstf, oidx4)
        return a4.reshape(N, EMB)

    g1 = _tc_mm1(x, W1, dis)
    agg1 = agg(g1)
    h1, g2 = _tc_mid(agg1, g1, dis, b1.reshape(1, EMB), W2,
                     jnp.zeros((N, EMB), jnp.float32),
                     use_res=False, use_relu=True, use_mm=True)
    agg2 = agg(g2)
    h2, g3 = _tc_mid(agg2, g2, dis, b2.reshape(1, EMB), W3, h1,
                     use_res=True, use_relu=True, use_mm=True)
    agg3 = agg(g3)
    out, _ = _tc_mid(agg3, g3, dis, b3.reshape(1, EMB), W3, h2,
                     use_res=True, use_relu=False, use_mm=False)
    return out


# speedup vs baseline: 23.2796x; 1.3105x over previous
import functools

import jax
import jax.numpy as jnp
from jax import lax
from jax.experimental import pallas as pl
from jax.experimental.pallas import tpu as pltpu
from jax.experimental.pallas import tpu_sc as plsc

N = 50000
E = 800000
EMB = 64
Q = 16
NQ = EMB // Q
K = 2
ROWS = E // 128
NT = 16
ZCH = 1000
NZC = N // ZCH
CCH = 125
NCC = N // CCH
DEGC = 2000
EC = 1000

_MESH = plsc.VectorSubcoreMesh(core_axis_name="c", subcore_axis_name="s")
_SC_PARAMS = pltpu.CompilerParams(use_tc_tiling_on_sc=False)


def _tile_row_range(sid):
    start = (sid * ROWS) // NT
    end = ((sid + 1) * ROWS) // NT
    return start, end


def _deg_body(dstf, deg_out, didx, ones_v, zbuf, dacc):
    cid = lax.axis_index("c")
    sid = lax.axis_index("s")

    def fill_ones(i, carry):
        ones_v[pl.ds(i * 16, 16)] = jnp.ones((16,), jnp.float32)
        return carry

    lax.fori_loop(0, DEGC // 16, fill_ones, 0)

    def fill_zero(i, carry):
        zbuf[pl.ds(i * 16, 16)] = jnp.zeros((16,), jnp.float32)
        return carry

    lax.fori_loop(0, DEGC // 16, fill_zero, 0)

    def zero_chunk(j, carry):
        @pl.when(j % NT == sid)
        def _():
            pltpu.sync_copy(zbuf, dacc.at[pl.ds(j * DEGC, DEGC)])
        return carry

    lax.fori_loop(0, N // DEGC, zero_chunk, 0)
    plsc.subcore_barrier()

    ebase = sid * (E // NT)

    def edge_chunk(i, carry):
        pltpu.sync_copy(dstf.at[pl.ds(ebase + i * DEGC, DEGC)], didx)
        pltpu.sync_copy(ones_v, dacc.at[didx], add=True)
        return carry

    lax.fori_loop(0, E // NT // DEGC, edge_chunk, 0)
    plsc.subcore_barrier()

    def copy_chunk(j, carry):
        @pl.when(jnp.logical_and(j % NT == sid, cid == 0))
        def _():
            pltpu.sync_copy(dacc.at[pl.ds(j * DEGC, DEGC)], zbuf)
            pltpu.sync_copy(zbuf, deg_out.at[pl.ds(j * DEGC, DEGC)])
        return carry

    lax.fori_loop(0, N // DEGC, copy_chunk, 0)


_deg_call = pl.kernel(
    _deg_body,
    out_type=jax.ShapeDtypeStruct((N,), jnp.float32),
    mesh=_MESH,
    compiler_params=_SC_PARAMS,
    scratch_types=[
        pltpu.VMEM((DEGC,), jnp.int32),
        pltpu.VMEM((DEGC,), jnp.float32),
        pltpu.VMEM((DEGC,), jnp.float32),
        pltpu.VMEM_SHARED((N,), jnp.float32),
    ],
)


def _agg_body(g4, src4, dstf, oidx4, agg_out,
              sidx0, sidx1, didx0, didx1, rows0, rows1, oidx,
              acc, isem, gsem, ssem):
    cid = lax.axis_index("c")
    sid = lax.axis_index("s")
    sidx = [sidx0, sidx1]
    didx = [didx0, didx1]
    rows = [rows0, rows1]

    ebase = sid * (E // NT)
    nit = E // NT // EC // K

    for q in range(2):
        tq = cid * 2 + q

        def fill_zero(i, carry):
            rows[1][i, :] = jnp.zeros((Q,), jnp.float32)
            return carry

        lax.fori_loop(0, ZCH, fill_zero, 0)

        def zero_chunk(j, carry):
            @pl.when(j % NT == sid)
            def _():
                pltpu.sync_copy(rows[1], acc.at[pl.ds(j * ZCH, ZCH)])
            return carry

        lax.fori_loop(0, NZC, zero_chunk, 0)
        plsc.subcore_barrier()

        def edge_iter(j, carry):
            base = ebase + j * (K * EC)
            di = []
            for k in range(K):
                sl = pl.ds(base + k * EC, EC)
                di.append(pltpu.async_copy(src4.at[tq].at[sl], sidx[k], isem))
                di.append(pltpu.async_copy(dstf.at[sl], didx[k], isem))
            dg = []
            for k in range(K):
                di[2 * k].wait()
                di[2 * k + 1].wait()
                dg.append(pltpu.async_copy(g4.at[sidx[k]], rows[k], gsem))
            ds = []
            for k in range(K):
                dg[k].wait()
                ds.append(pltpu.async_copy(rows[k], acc.at[didx[k]], ssem,
                                           add=True))
            for k in range(K):
                ds[k].wait()
            return carry

        lax.fori_loop(0, nit, edge_iter, 0)
        plsc.subcore_barrier()

        def copy_chunk(j, carry):
            @pl.when(j % NT == sid)
            def _():
                pltpu.sync_copy(acc.at[pl.ds(j * CCH, CCH)],
                                rows[0].at[pl.ds(0, CCH)])
                pltpu.sync_copy(oidx4.at[tq].at[j], oidx)
                pltpu.sync_copy(rows[0].at[pl.ds(0, CCH)], agg_out.at[oidx])
            return carry

        lax.fori_loop(0, NCC, copy_chunk, 0)
        plsc.subcore_barrier()


_agg_call = pl.kernel(
    _agg_body,
    out_type=jax.ShapeDtypeStruct((NQ * N, Q), jnp.float32),
    mesh=_MESH,
    compiler_params=_SC_PARAMS,
    scratch_types=(
        [pltpu.VMEM((EC,), jnp.int32)] * 4
        + [pltpu.VMEM((EC, Q), jnp.float32)] * 2
        + [
            pltpu.VMEM((CCH,), jnp.int32),
            pltpu.VMEM_SHARED((N, Q), jnp.float32),
            pltpu.SemaphoreType.DMA,
            pltpu.SemaphoreType.DMA,
            pltpu.SemaphoreType.DMA,
        ]
    ),
)


BLK = 2000
GRID = N // BLK


def _dis_kernel(deg_ref, dis_ref):
    dis_ref[...] = lax.rsqrt(deg_ref[...] + 1.0)


def _tc_dis(deg):
    return pl.pallas_call(
        _dis_kernel,
        out_shape=jax.ShapeDtypeStruct((GRID, BLK), jnp.float32),
        grid=(1,),
        in_specs=[pl.BlockSpec((GRID, BLK), lambda i: (0, 0))],
        out_specs=pl.BlockSpec((GRID, BLK), lambda i: (0, 0)),
    )(deg.reshape(GRID, BLK))


def _src4_kernel(src_ref, out_ref):
    s = src_ref[...] * NQ
    out_ref[...] = jnp.stack([s + q for q in range(NQ)], axis=0)


def _tc_src4(src2):
    return pl.pallas_call(
        _src4_kernel,
        out_shape=jax.ShapeDtypeStruct((NQ, ROWS, 128), jnp.int32),
        grid=(1,),
        in_specs=[pl.BlockSpec((ROWS, 128), lambda i: (0, 0))],
        out_specs=pl.BlockSpec((NQ, ROWS, 128), lambda i: (0, 0, 0)),
    )(src2)


def _oidx_kernel(out_ref):
    qi = lax.broadcasted_iota(jnp.int32, (NQ, NCC, CCH), 0)
    ri = lax.broadcasted_iota(jnp.int32, (NQ, NCC, CCH), 1)
    ci = lax.broadcasted_iota(jnp.int32, (NQ, NCC, CCH), 2)
    out_ref[...] = NQ * (ri * CCH + ci) + qi


def _tc_oidx():
    return pl.pallas_call(
        _oidx_kernel,
        out_shape=jax.ShapeDtypeStruct((NQ, NCC, CCH), jnp.int32),
        grid=(1,),
        out_specs=pl.BlockSpec((NQ, NCC, CCH), lambda i: (0, 0, 0)),
    )()


def _mm1_kernel(x_ref, w_ref, dis_ref, g_ref):
    y = jnp.dot(x_ref[...], w_ref[...], preferred_element_type=jnp.float32)
    g_ref[...] = y * dis_ref[...]


def _tc_mm1(x, W1, dis):
    return pl.pallas_call(
        _mm1_kernel,
        out_shape=jax.ShapeDtypeStruct((N, EMB), jnp.float32),
        grid=(GRID,),
        in_specs=[
            pl.BlockSpec((BLK, 8), lambda i: (i, 0)),
            pl.BlockSpec((8, EMB), lambda i: (0, 0)),
            pl.BlockSpec((BLK, 1), lambda i: (i, 0)),
        ],
        out_specs=pl.BlockSpec((BLK, EMB), lambda i: (i, 0)),
    )(x, W1, dis)


def _mid_kernel(agg_ref, g_ref, dis_ref, b_ref, w_ref, vp_ref,
                v_ref, gn_ref):
    dis = dis_ref[...]
    v = dis * (agg_ref[...] + g_ref[...]) + b_ref[...] \
        + jnp.maximum(vp_ref[...], 0.0)
    v_ref[...] = v
    h = jnp.maximum(v, 0.0)
    gn_ref[...] = jnp.dot(h, w_ref[...],
                          preferred_element_type=jnp.float32) * dis


def _tc_mid(agg, g, dis, b, W, vprev):
    return pl.pallas_call(
        _mid_kernel,
        out_shape=[
            jax.ShapeDtypeStruct((N, EMB), jnp.float32),
            jax.ShapeDtypeStruct((N, EMB), jnp.float32),
        ],
        grid=(GRID,),
        in_specs=[
            pl.BlockSpec((BLK, EMB), lambda i: (i, 0)),
            pl.BlockSpec((BLK, EMB), lambda i: (i, 0)),
            pl.BlockSpec((BLK, 1), lambda i: (i, 0)),
            pl.BlockSpec((1, EMB), lambda i: (0, 0)),
            pl.BlockSpec((EMB, EMB), lambda i: (0, 0)),
            pl.BlockSpec((BLK, EMB), lambda i: (i, 0)),
        ],
        out_specs=[
            pl.BlockSpec((BLK, EMB), lambda i: (i, 0)),
            pl.BlockSpec((BLK, EMB), lambda i: (i, 0)),
        ],
    )(agg, g, dis, b, W, vprev)


@jax.jit
def kernel(x, edge_index, W1, b1, W2, b2, W3, b3):
    src2 = edge_index[0].reshape(ROWS, 128)
    dstf = edge_index[1]

    deg = _deg_call(dstf)
    dis = _tc_dis(deg).reshape(N, 1)
    src2x = _tc_src4(src2).reshape(NQ, E)
    oidx2 = _tc_oidx()

    g1 = _tc_mm1(x, W1, dis)

    def layer(g, vprev, W, b):
        a = _agg_call(g.reshape(NQ * N, Q), src2x, dstf,
                      oidx2).reshape(N, EMB)
        return _tc_mid(a, g, dis, b.reshape(1, EMB), W, vprev)

    v1, g2 = layer(g1, jnp.zeros((N, EMB), jnp.float32), W2, b1)
    v2, g3 = layer(g2, v1, W3, b2)
    v3, _ = layer(g3, v2, W3, b3)
    return v3


# speedup vs baseline: 25.6631x; 1.1024x over previous
import functools

import jax
import jax.numpy as jnp
from jax import lax
from jax.experimental import pallas as pl
from jax.experimental.pallas import tpu as pltpu
from jax.experimental.pallas import tpu_sc as plsc

N = 50000
E = 800000
EMB = 64
Q = 16
NQ = EMB // Q
K = 2
ROWS = E // 128
NT = 16
ZCH = 1000
NZC = N // ZCH
CCH = 125
NCC = N // CCH
DEGC = 2000
EC = 1000

_MESH = plsc.VectorSubcoreMesh(core_axis_name="c", subcore_axis_name="s")
_SC_PARAMS = pltpu.CompilerParams(use_tc_tiling_on_sc=False)


def _tile_row_range(sid):
    start = (sid * ROWS) // NT
    end = ((sid + 1) * ROWS) // NT
    return start, end


def _deg_body(dstf, deg_out, didx, ones_v, zbuf, dacc):
    cid = lax.axis_index("c")
    sid = lax.axis_index("s")

    def fill_ones(i, carry):
        ones_v[pl.ds(i * 16, 16)] = jnp.ones((16,), jnp.float32)
        return carry

    lax.fori_loop(0, DEGC // 16, fill_ones, 0)

    def fill_zero(i, carry):
        zbuf[pl.ds(i * 16, 16)] = jnp.zeros((16,), jnp.float32)
        return carry

    lax.fori_loop(0, DEGC // 16, fill_zero, 0)

    def zero_chunk(j, carry):
        @pl.when(j % NT == sid)
        def _():
            pltpu.sync_copy(zbuf, dacc.at[pl.ds(j * DEGC, DEGC)])
        return carry

    lax.fori_loop(0, N // DEGC, zero_chunk, 0)
    plsc.subcore_barrier()

    ebase = sid * (E // NT)

    def edge_chunk(i, carry):
        pltpu.sync_copy(dstf.at[pl.ds(ebase + i * DEGC, DEGC)], didx)
        pltpu.sync_copy(ones_v, dacc.at[didx], add=True)
        return carry

    lax.fori_loop(0, E // NT // DEGC, edge_chunk, 0)
    plsc.subcore_barrier()

    def copy_chunk(j, carry):
        @pl.when(jnp.logical_and(j % NT == sid, cid == 0))
        def _():
            pltpu.sync_copy(dacc.at[pl.ds(j * DEGC, DEGC)], zbuf)
            pltpu.sync_copy(zbuf, deg_out.at[pl.ds(j * DEGC, DEGC)])
        return carry

    lax.fori_loop(0, N // DEGC, copy_chunk, 0)


_deg_call = pl.kernel(
    _deg_body,
    out_type=jax.ShapeDtypeStruct((N,), jnp.float32),
    mesh=_MESH,
    compiler_params=_SC_PARAMS,
    scratch_types=[
        pltpu.VMEM((DEGC,), jnp.int32),
        pltpu.VMEM((DEGC,), jnp.float32),
        pltpu.VMEM((DEGC,), jnp.float32),
        pltpu.VMEM_SHARED((N,), jnp.float32),
    ],
)


GC = 10
NGRP = E // NT // EC // GC


def _agg_body(g4, src4, dstf3, oidx4, agg_out,
              sidxb, didxb, rows0, rows1, oidx,
              acc, isem, gsem, ssem):
    cid = lax.axis_index("c")
    sid = lax.axis_index("s")
    rows = [rows0, rows1]

    for q in range(2):
        tq = cid * 2 + q

        def fill_zero(i, carry):
            rows[1][i, :] = jnp.zeros((Q,), jnp.float32)
            return carry

        lax.fori_loop(0, ZCH, fill_zero, 0)

        def zero_chunk(j, carry):
            @pl.when(j % NT == sid)
            def _():
                pltpu.sync_copy(rows[1], acc.at[pl.ds(j * ZCH, ZCH)])
            return carry

        lax.fori_loop(0, NZC, zero_chunk, 0)
        plsc.subcore_barrier()

        rbase = sid * (E // NT // EC)

        def edge_group(j, carry):
            @pl.when(j > 0)
            def _():
                pltpu.make_async_copy(rows[0],
                                      acc.at[didxb.at[GC - 2]], ssem).wait()
                pltpu.make_async_copy(rows[1],
                                      acc.at[didxb.at[GC - 1]], ssem).wait()
            gsl = pl.ds(rbase + j * GC, GC)
            ia = pltpu.async_copy(src4.at[tq].at[gsl], sidxb, isem)
            ib = pltpu.async_copy(dstf3.at[gsl], didxb, isem)
            ia.wait()
            ib.wait()
            gd = [None] * GC
            sd = [None] * GC
            for k in range(GC):
                if k >= 2:
                    sd[k - 2].wait()
                gd[k] = pltpu.async_copy(g4.at[sidxb.at[k]], rows[k % 2],
                                         gsem)
                if k >= 1:
                    gd[k - 1].wait()
                    sd[k - 1] = pltpu.async_copy(
                        rows[(k - 1) % 2], acc.at[didxb.at[k - 1]], ssem,
                        add=True)
            gd[GC - 1].wait()
            sd[GC - 1] = pltpu.async_copy(
                rows[(GC - 1) % 2], acc.at[didxb.at[GC - 1]], ssem, add=True)
            return carry

        lax.fori_loop(0, NGRP, edge_group, 0)
        pltpu.make_async_copy(rows[0], acc.at[didxb.at[GC - 2]], ssem).wait()
        pltpu.make_async_copy(rows[1], acc.at[didxb.at[GC - 1]], ssem).wait()
        plsc.subcore_barrier()

        def copy_chunk(j, carry):
            @pl.when(j % NT == sid)
            def _():
                pltpu.sync_copy(acc.at[pl.ds(j * CCH, CCH)],
                                rows[0].at[pl.ds(0, CCH)])
                pltpu.sync_copy(oidx4.at[tq].at[j], oidx)
                pltpu.sync_copy(rows[0].at[pl.ds(0, CCH)], agg_out.at[oidx])
            return carry

        lax.fori_loop(0, NCC, copy_chunk, 0)
        plsc.subcore_barrier()


_agg_call = pl.kernel(
    _agg_body,
    out_type=jax.ShapeDtypeStruct((NQ * N, Q), jnp.float32),
    mesh=_MESH,
    compiler_params=_SC_PARAMS,
    scratch_types=[
        pltpu.VMEM((GC, EC), jnp.int32),
        pltpu.VMEM((GC, EC), jnp.int32),
        pltpu.VMEM((EC, Q), jnp.float32),
        pltpu.VMEM((EC, Q), jnp.float32),
        pltpu.VMEM((CCH,), jnp.int32),
        pltpu.VMEM_SHARED((N, Q), jnp.float32),
        pltpu.SemaphoreType.DMA,
        pltpu.SemaphoreType.DMA,
        pltpu.SemaphoreType.DMA,
    ],
)


BLK = 2000
GRID = N // BLK


def _dis_kernel(deg_ref, dis_ref):
    dis_ref[...] = lax.rsqrt(deg_ref[...] + 1.0)


def _tc_dis(deg):
    return pl.pallas_call(
        _dis_kernel,
        out_shape=jax.ShapeDtypeStruct((GRID, BLK), jnp.float32),
        grid=(1,),
        in_specs=[pl.BlockSpec((GRID, BLK), lambda i: (0, 0))],
        out_specs=pl.BlockSpec((GRID, BLK), lambda i: (0, 0)),
    )(deg.reshape(GRID, BLK))


def _src4_kernel(src_ref, out_ref):
    s = src_ref[...] * NQ
    out_ref[...] = jnp.stack([s + q for q in range(NQ)], axis=0)


def _tc_src4(src2):
    return pl.pallas_call(
        _src4_kernel,
        out_shape=jax.ShapeDtypeStruct((NQ, ROWS, 128), jnp.int32),
        grid=(1,),
        in_specs=[pl.BlockSpec((ROWS, 128), lambda i: (0, 0))],
        out_specs=pl.BlockSpec((NQ, ROWS, 128), lambda i: (0, 0, 0)),
    )(src2)


def _oidx_kernel(out_ref):
    qi = lax.broadcasted_iota(jnp.int32, (NQ, NCC, CCH), 0)
    ri = lax.broadcasted_iota(jnp.int32, (NQ, NCC, CCH), 1)
    ci = lax.broadcasted_iota(jnp.int32, (NQ, NCC, CCH), 2)
    out_ref[...] = NQ * (ri * CCH + ci) + qi


def _tc_oidx():
    return pl.pallas_call(
        _oidx_kernel,
        out_shape=jax.ShapeDtypeStruct((NQ, NCC, CCH), jnp.int32),
        grid=(1,),
        out_specs=pl.BlockSpec((NQ, NCC, CCH), lambda i: (0, 0, 0)),
    )()


def _mm1_kernel(x_ref, w_ref, dis_ref, g_ref):
    y = jnp.dot(x_ref[...], w_ref[...], preferred_element_type=jnp.float32)
    g_ref[...] = y * dis_ref[...]


def _tc_mm1(x, W1, dis):
    return pl.pallas_call(
        _mm1_kernel,
        out_shape=jax.ShapeDtypeStruct((N, EMB), jnp.float32),
        grid=(GRID,),
        in_specs=[
            pl.BlockSpec((BLK, 8), lambda i: (i, 0)),
            pl.BlockSpec((8, EMB), lambda i: (0, 0)),
            pl.BlockSpec((BLK, 1), lambda i: (i, 0)),
        ],
        out_specs=pl.BlockSpec((BLK, EMB), lambda i: (i, 0)),
    )(x, W1, dis)


def _mid_kernel(agg_ref, g_ref, dis_ref, b_ref, w_ref, vp_ref,
                v_ref, gn_ref):
    dis = dis_ref[...]
    v = dis * (agg_ref[...] + g_ref[...]) + b_ref[...] \
        + jnp.maximum(vp_ref[...], 0.0)
    v_ref[...] = v
    h = jnp.maximum(v, 0.0)
    gn_ref[...] = jnp.dot(h, w_ref[...],
                          preferred_element_type=jnp.float32) * dis


def _tc_mid(agg, g, dis, b, W, vprev):
    return pl.pallas_call(
        _mid_kernel,
        out_shape=[
            jax.ShapeDtypeStruct((N, EMB), jnp.float32),
            jax.ShapeDtypeStruct((N, EMB), jnp.float32),
        ],
        grid=(GRID,),
        in_specs=[
            pl.BlockSpec((BLK, EMB), lambda i: (i, 0)),
            pl.BlockSpec((BLK, EMB), lambda i: (i, 0)),
            pl.BlockSpec((BLK, 1), lambda i: (i, 0)),
            pl.BlockSpec((1, EMB), lambda i: (0, 0)),
            pl.BlockSpec((EMB, EMB), lambda i: (0, 0)),
            pl.BlockSpec((BLK, EMB), lambda i: (i, 0)),
        ],
        out_specs=[
            pl.BlockSpec((BLK, EMB), lambda i: (i, 0)),
            pl.BlockSpec((BLK, EMB), lambda i: (i, 0)),
        ],
    )(agg, g, dis, b, W, vprev)


@jax.jit
def kernel(x, edge_index, W1, b1, W2, b2, W3, b3):
    src2 = edge_index[0].reshape(ROWS, 128)
    dstf = edge_index[1]

    deg = _deg_call(dstf)
    dis = _tc_dis(deg).reshape(N, 1)
    src4g = _tc_src4(src2).reshape(NQ, E // EC, EC)
    dst3 = edge_index[1].reshape(E // EC, EC)
    oidx4v = _tc_oidx()

    g1 = _tc_mm1(x, W1, dis)

    def layer(g, vprev, W, b):
        a = _agg_call(g.reshape(NQ * N, Q), src4g, dst3,
                      oidx4v).reshape(N, EMB)
        return _tc_mid(a, g, dis, b.reshape(1, EMB), W, vprev)

    v1, g2 = layer(g1, jnp.zeros((N, EMB), jnp.float32), W2, b1)
    v2, g3 = layer(g2, v1, W3, b2)
    v3, _ = layer(g3, v2, W3, b3)
    return v3


# speedup vs baseline: 27.6540x; 1.0776x over previous
import functools

import jax
import jax.numpy as jnp
from jax import lax
from jax.experimental import pallas as pl
from jax.experimental.pallas import tpu as pltpu
from jax.experimental.pallas import tpu_sc as plsc

N = 50000
E = 800000
EMB = 64
Q = 16
NQ = EMB // Q
K = 2
ROWS = E // 128
NT = 16
ZCH = 1000
NZC = N // ZCH
CCH = 1000
NCC = N // CCH
DEGC = 2000
EC = 1000

_MESH = plsc.VectorSubcoreMesh(core_axis_name="c", subcore_axis_name="s")
_SC_PARAMS = pltpu.CompilerParams(use_tc_tiling_on_sc=False)


def _tile_row_range(sid):
    start = (sid * ROWS) // NT
    end = ((sid + 1) * ROWS) // NT
    return start, end


def _deg_body(dstf, deg_out, didx, ones_v, zbuf, dacc):
    cid = lax.axis_index("c")
    sid = lax.axis_index("s")

    def fill_ones(i, carry):
        ones_v[pl.ds(i * 16, 16)] = jnp.ones((16,), jnp.float32)
        return carry

    lax.fori_loop(0, DEGC // 16, fill_ones, 0)

    def fill_zero(i, carry):
        zbuf[pl.ds(i * 16, 16)] = jnp.zeros((16,), jnp.float32)
        return carry

    lax.fori_loop(0, DEGC // 16, fill_zero, 0)

    def zero_chunk(j, carry):
        @pl.when(j % NT == sid)
        def _():
            pltpu.sync_copy(zbuf, dacc.at[pl.ds(j * DEGC, DEGC)])
        return carry

    lax.fori_loop(0, N // DEGC, zero_chunk, 0)
    plsc.subcore_barrier()

    ebase = sid * (E // NT)

    def edge_chunk(i, carry):
        pltpu.sync_copy(dstf.at[pl.ds(ebase + i * DEGC, DEGC)], didx)
        pltpu.sync_copy(ones_v, dacc.at[didx], add=True)
        return carry

    lax.fori_loop(0, E // NT // DEGC, edge_chunk, 0)
    plsc.subcore_barrier()

    def copy_chunk(j, carry):
        @pl.when(jnp.logical_and(j % NT == sid, cid == 0))
        def _():
            pltpu.sync_copy(dacc.at[pl.ds(j * DEGC, DEGC)], zbuf)
            pltpu.sync_copy(zbuf, deg_out.at[pl.ds(j * DEGC, DEGC)])
        return carry

    lax.fori_loop(0, N // DEGC, copy_chunk, 0)


_deg_call = pl.kernel(
    _deg_body,
    out_type=jax.ShapeDtypeStruct((N,), jnp.float32),
    mesh=_MESH,
    compiler_params=_SC_PARAMS,
    scratch_types=[
        pltpu.VMEM((DEGC,), jnp.int32),
        pltpu.VMEM((DEGC,), jnp.float32),
        pltpu.VMEM((DEGC,), jnp.float32),
        pltpu.VMEM_SHARED((N,), jnp.float32),
    ],
)


GC = 10
NGRP = E // NT // EC // GC


def _agg_body(g4, src4, dstf3, oidx4, agg_out,
              sidxb, didxb, rows0, rows1, oidx,
              acc, isem, gsem, ssem):
    cid = lax.axis_index("c")
    sid = lax.axis_index("s")
    rows = [rows0, rows1]

    for q in range(2):
        tq = cid * 2 + q

        def fill_zero(i, carry):
            rows[1][i, :] = jnp.zeros((Q,), jnp.float32)
            return carry

        lax.fori_loop(0, ZCH, fill_zero, 0)

        def zero_chunk(j, carry):
            @pl.when(j % NT == sid)
            def _():
                pltpu.sync_copy(rows[1], acc.at[pl.ds(j * ZCH, ZCH)])
            return carry

        lax.fori_loop(0, NZC, zero_chunk, 0)
        plsc.subcore_barrier()

        rbase = sid * (E // NT // EC)

        def edge_group(j, carry):
            @pl.when(j > 0)
            def _():
                pltpu.make_async_copy(rows[0],
                                      acc.at[didxb.at[GC - 2]], ssem).wait()
                pltpu.make_async_copy(rows[1],
                                      acc.at[didxb.at[GC - 1]], ssem).wait()
            gsl = pl.ds(rbase + j * GC, GC)
            ia = pltpu.async_copy(src4.at[tq].at[gsl], sidxb, isem)
            ib = pltpu.async_copy(dstf3.at[gsl], didxb, isem)
            ia.wait()
            ib.wait()
            gd = [None] * GC
            sd = [None] * GC
            for k in range(GC):
                if k >= 2:
                    sd[k - 2].wait()
                gd[k] = pltpu.async_copy(g4.at[sidxb.at[k]], rows[k % 2],
                                         gsem)
                if k >= 1:
                    gd[k - 1].wait()
                    sd[k - 1] = pltpu.async_copy(
                        rows[(k - 1) % 2], acc.at[didxb.at[k - 1]], ssem,
                        add=True)
            gd[GC - 1].wait()
            sd[GC - 1] = pltpu.async_copy(
                rows[(GC - 1) % 2], acc.at[didxb.at[GC - 1]], ssem, add=True)
            return carry

        lax.fori_loop(0, NGRP, edge_group, 0)
        pltpu.make_async_copy(rows[0], acc.at[didxb.at[GC - 2]], ssem).wait()
        pltpu.make_async_copy(rows[1], acc.at[didxb.at[GC - 1]], ssem).wait()
        plsc.subcore_barrier()

        def copy_chunk(j, carry):
            @pl.when(j % NT == sid)
            def _():
                pltpu.sync_copy(acc.at[pl.ds(j * CCH, CCH)], rows[0])
                pltpu.sync_copy(oidx4.at[tq].at[j], oidx)
                pltpu.sync_copy(rows[0], agg_out.at[oidx])
            return carry

        lax.fori_loop(0, NCC, copy_chunk, 0)
        plsc.subcore_barrier()


_agg_call = pl.kernel(
    _agg_body,
    out_type=jax.ShapeDtypeStruct((NQ * N, Q), jnp.float32),
    mesh=_MESH,
    compiler_params=_SC_PARAMS,
    scratch_types=[
        pltpu.VMEM((GC, EC), jnp.int32),
        pltpu.VMEM((GC, EC), jnp.int32),
        pltpu.VMEM((EC, Q), jnp.float32),
        pltpu.VMEM((EC, Q), jnp.float32),
        pltpu.VMEM((CCH,), jnp.int32),
        pltpu.VMEM_SHARED((N, Q), jnp.float32),
        pltpu.SemaphoreType.DMA,
        pltpu.SemaphoreType.DMA,
        pltpu.SemaphoreType.DMA,
    ],
)


BLK = 2000
GRID = N // BLK


def _dis_kernel(deg_ref, dis_ref):
    dis_ref[...] = lax.rsqrt(deg_ref[...] + 1.0)


def _tc_dis(deg):
    return pl.pallas_call(
        _dis_kernel,
        out_shape=jax.ShapeDtypeStruct((GRID, BLK), jnp.float32),
        grid=(1,),
        in_specs=[pl.BlockSpec((GRID, BLK), lambda i: (0, 0))],
        out_specs=pl.BlockSpec((GRID, BLK), lambda i: (0, 0)),
    )(deg.reshape(GRID, BLK))


def _src4_kernel(src_ref, out_ref):
    s = src_ref[...] * NQ
    out_ref[...] = jnp.stack([s + q for q in range(NQ)], axis=0)


def _tc_src4(src2):
    return pl.pallas_call(
        _src4_kernel,
        out_shape=jax.ShapeDtypeStruct((NQ, ROWS, 128), jnp.int32),
        grid=(1,),
        in_specs=[pl.BlockSpec((ROWS, 128), lambda i: (0, 0))],
        out_specs=pl.BlockSpec((NQ, ROWS, 128), lambda i: (0, 0, 0)),
    )(src2)


def _oidx_kernel(out_ref):
    qi = lax.broadcasted_iota(jnp.int32, (NQ, NCC, CCH), 0)
    ri = lax.broadcasted_iota(jnp.int32, (NQ, NCC, CCH), 1)
    ci = lax.broadcasted_iota(jnp.int32, (NQ, NCC, CCH), 2)
    out_ref[...] = NQ * (ri * CCH + ci) + qi


def _tc_oidx():
    return pl.pallas_call(
        _oidx_kernel,
        out_shape=jax.ShapeDtypeStruct((NQ, NCC, CCH), jnp.int32),
        grid=(1,),
        out_specs=pl.BlockSpec((NQ, NCC, CCH), lambda i: (0, 0, 0)),
    )()


def _mm1_kernel(x_ref, w_ref, dis_ref, g_ref):
    y = jnp.dot(x_ref[...], w_ref[...], preferred_element_type=jnp.float32)
    g_ref[...] = y * dis_ref[...]


def _tc_mm1(x, W1, dis):
    return pl.pallas_call(
        _mm1_kernel,
        out_shape=jax.ShapeDtypeStruct((N, EMB), jnp.float32),
        grid=(GRID,),
        in_specs=[
            pl.BlockSpec((BLK, 8), lambda i: (i, 0)),
            pl.BlockSpec((8, EMB), lambda i: (0, 0)),
            pl.BlockSpec((BLK, 1), lambda i: (i, 0)),
        ],
        out_specs=pl.BlockSpec((BLK, EMB), lambda i: (i, 0)),
    )(x, W1, dis)


def _mid_kernel(agg_ref, g_ref, dis_ref, b_ref, w_ref, vp_ref,
                v_ref, gn_ref):
    dis = dis_ref[...]
    v = dis * (agg_ref[...] + g_ref[...]) + b_ref[...] \
        + jnp.maximum(vp_ref[...], 0.0)
    v_ref[...] = v
    h = jnp.maximum(v, 0.0)
    gn_ref[...] = jnp.dot(h, w_ref[...],
                          preferred_element_type=jnp.float32) * dis


def _tc_mid(agg, g, dis, b, W, vprev):
    return pl.pallas_call(
        _mid_kernel,
        out_shape=[
            jax.ShapeDtypeStruct((N, EMB), jnp.float32),
            jax.ShapeDtypeStruct((N, EMB), jnp.float32),
        ],
        grid=(GRID,),
        in_specs=[
            pl.BlockSpec((BLK, EMB), lambda i: (i, 0)),
            pl.BlockSpec((BLK, EMB), lambda i: (i, 0)),
            pl.BlockSpec((BLK, 1), lambda i: (i, 0)),
            pl.BlockSpec((1, EMB), lambda i: (0, 0)),
            pl.BlockSpec((EMB, EMB), lambda i: (0, 0)),
            pl.BlockSpec((BLK, EMB), lambda i: (i, 0)),
        ],
        out_specs=[
            pl.BlockSpec((BLK, EMB), lambda i: (i, 0)),
            pl.BlockSpec((BLK, EMB), lambda i: (i, 0)),
        ],
    )(agg, g, dis, b, W, vprev)


@jax.jit
def kernel(x, edge_index, W1, b1, W2, b2, W3, b3):
    src2 = edge_index[0].reshape(ROWS, 128)
    dstf = edge_index[1]

    deg = _deg_call(dstf)
    dis = _tc_dis(deg).reshape(N, 1)
    src4g = _tc_src4(src2).reshape(NQ, E // EC, EC)
    dst3 = edge_index[1].reshape(E // EC, EC)
    oidx4v = _tc_oidx()

    g1 = _tc_mm1(x, W1, dis)

    def layer(g, vprev, W, b):
        a = _agg_call(g.reshape(NQ * N, Q), src4g, dst3,
                      oidx4v).reshape(N, EMB)
        return _tc_mid(a, g, dis, b.reshape(1, EMB), W, vprev)

    v1, g2 = layer(g1, jnp.zeros((N, EMB), jnp.float32), W2, b1)
    v2, g3 = layer(g2, v1, W3, b2)
    v3, _ = layer(g3, v2, W3, b3)
    return v3
